# baseline (speedup 1.0000x reference)
import jax
import jax.numpy as jnp
from jax.experimental import pallas as pl
from jax.experimental.pallas import tpu as pltpu

_C = 512
_NBUF = 4


def _body(x_ref, t_ref, o_ref, tbuf, xbuf, obuf, tsems, xsems, osems):
    S = t_ref.shape[0]
    N = x_ref.shape[0] // _C
    NT = S // _C

    def tcopy(j):
        return pltpu.make_async_copy(
            t_ref.at[pl.ds(j * _C, _C)], tbuf.at[pl.ds(j * _C, _C)],
            tsems.at[j])

    def xcopy(i):
        return pltpu.make_async_copy(
            x_ref.at[pl.ds(i * _C, _C)], xbuf.at[i % _NBUF],
            xsems.at[i % _NBUF])

    def ocopy(i):
        return pltpu.make_async_copy(
            obuf.at[i % _NBUF], o_ref.at[pl.ds(i * _C, _C)],
            osems.at[i % _NBUF])

    for j in range(NT):
        tcopy(j).start()
    for i in range(_NBUF):
        xcopy(i).start()

    for i in range(N):
        if i < NT:
            tcopy(i).wait()
        if i >= _NBUF:
            ocopy(i - _NBUF).wait()
        xcopy(i).wait()
        slot = i % _NBUF
        trow = (i % NT) * _C
        obuf[slot] = xbuf[slot] + tbuf[pl.ds(trow, _C)]
        ocopy(i).start()
        if i + _NBUF < N:
            xcopy(i + _NBUF).start()

    for i in range(max(0, N - _NBUF), N):
        ocopy(i).wait()


def kernel(x, embed_table):
    B, S, D = x.shape
    x2 = x.reshape(B * S, D)
    out = pl.pallas_call(
        _body,
        in_specs=[
            pl.BlockSpec(memory_space=pl.ANY),
            pl.BlockSpec(memory_space=pl.ANY),
        ],
        out_specs=pl.BlockSpec(memory_space=pl.ANY),
        out_shape=jax.ShapeDtypeStruct((B * S, D), x.dtype),
        scratch_shapes=[
            pltpu.VMEM((S, D), x.dtype),
            pltpu.VMEM((_NBUF, _C, D), x.dtype),
            pltpu.VMEM((_NBUF, _C, D), x.dtype),
            pltpu.SemaphoreType.DMA((S // _C,)),
            pltpu.SemaphoreType.DMA((_NBUF,)),
            pltpu.SemaphoreType.DMA((_NBUF,)),
        ],
    )(x2, embed_table)
    return out.reshape(B, S, D)
